# baseline (speedup 1.0000x reference)
import jax
import jax.numpy as jnp
from jax import lax
from jax.experimental import pallas as pl
from jax.experimental.pallas import tpu as pltpu
from jax.experimental.pallas import tpu_sc as plsc

BATCH = 16384
HIST_LEN = 50
EMBEDDING_K = 32

_NW = 32
_CB = BATCH // _NW
_L = 16


def _transpose_block(rows_v, trans_v):
    iota = lax.iota(jnp.int32, _L)
    cols = [jnp.full((_L,), k, dtype=jnp.int32) for k in range(EMBEDDING_K)]

    @plsc.parallel_loop(0, _CB, _L, unroll=4)
    def _(j0):
        block = rows_v.at[pl.ds(j0, _L)]
        for k in range(EMBEDDING_K):
            vals = plsc.load_gather(block, [iota, cols[k]])
            trans_v[k, pl.ds(j0, _L)] = vals


def _gather_kernel(table_hbm, idx_hbm, out_hbm,
                   idx_a, idx_b, rows_a, rows_b, trans_a, trans_b,
                   gsem_a, gsem_b, osem_a, osem_b):
    wid = lax.axis_index("s") * 2 + lax.axis_index("c")
    b0 = wid * _CB

    idx_bufs = (idx_a, idx_b)
    row_bufs = (rows_a, rows_b)
    trans_bufs = (trans_a, trans_b)
    gsems = (gsem_a, gsem_b)
    osems = (osem_a, osem_b)

    pltpu.sync_copy(idx_hbm.at[0, pl.ds(b0, _CB)], idx_a)
    pltpu.async_copy(table_hbm.at[idx_a], rows_a, gsem_a)

    @pl.loop(0, HIST_LEN, step=2)
    def _(l0):
        for p in range(2):
            l = l0 + p
            cur = p
            nxt = 1 - p

            @pl.when(l + 1 < HIST_LEN)
            def _():
                pltpu.sync_copy(idx_hbm.at[l + 1, pl.ds(b0, _CB)],
                                idx_bufs[nxt])
                pltpu.async_copy(table_hbm.at[idx_bufs[nxt]],
                                 row_bufs[nxt], gsems[nxt])

            pltpu.make_async_copy(table_hbm.at[idx_bufs[cur]],
                                  row_bufs[cur], gsems[cur]).wait()

            @pl.when(l >= 2)
            def _():
                pltpu.make_async_copy(
                    trans_bufs[cur],
                    out_hbm.at[l - 2, :, pl.ds(b0, _CB)],
                    osems[cur],
                ).wait()

            _transpose_block(row_bufs[cur], trans_bufs[cur])
            pltpu.async_copy(trans_bufs[cur],
                             out_hbm.at[l, :, pl.ds(b0, _CB)],
                             osems[cur])

    for l in (HIST_LEN - 2, HIST_LEN - 1):
        pltpu.make_async_copy(
            trans_bufs[l % 2],
            out_hbm.at[l, :, pl.ds(b0, _CB)],
            osems[l % 2],
        ).wait()


@jax.jit
def _sc_gather(item_table, idx_t):
    mesh = plsc.VectorSubcoreMesh(core_axis_name="c", subcore_axis_name="s")
    return pl.kernel(
        _gather_kernel,
        out_type=jax.ShapeDtypeStruct((HIST_LEN, EMBEDDING_K, BATCH),
                                      jnp.float32),
        mesh=mesh,
        compiler_params=pltpu.CompilerParams(use_tc_tiling_on_sc=False,
                                             needs_layout_passes=False),
        scratch_types=[
            pltpu.VMEM((_CB,), jnp.int32),
            pltpu.VMEM((_CB,), jnp.int32),
            pltpu.VMEM((_CB, EMBEDDING_K), jnp.float32),
            pltpu.VMEM((_CB, EMBEDDING_K), jnp.float32),
            pltpu.VMEM((EMBEDDING_K, _CB), jnp.float32),
            pltpu.VMEM((EMBEDDING_K, _CB), jnp.float32),
            pltpu.SemaphoreType.DMA,
            pltpu.SemaphoreType.DMA,
            pltpu.SemaphoreType.DMA,
            pltpu.SemaphoreType.DMA,
        ],
    )(item_table, idx_t)


def kernel(item_table, item_idx):
    idx_t = item_idx.T.astype(jnp.int32)
    out_t = _sc_gather(item_table, idx_t)
    return out_t.transpose(2, 0, 1)


# speedup vs baseline: 1.1625x; 1.1625x over previous
import jax
import jax.numpy as jnp
from jax import lax
from jax.experimental import pallas as pl
from jax.experimental.pallas import tpu as pltpu
from jax.experimental.pallas import tpu_sc as plsc

BATCH = 16384
HIST_LEN = 50
EMBEDDING_K = 32

_NW = 32
_CB = BATCH // _NW
_L = 16


def _transpose_block(rows_v, trans_v):
    iota = lax.iota(jnp.int32, _L)
    cols = [jnp.full((_L,), k, dtype=jnp.int32) for k in range(EMBEDDING_K)]

    @plsc.parallel_loop(0, _CB, _L, unroll=2)
    def _(j0):
        block = rows_v.at[pl.ds(j0, _L)]
        bgl = j0 // 128
        br = j0 % 128
        for k in range(EMBEDDING_K):
            vals = plsc.load_gather(block, [iota, cols[k]])
            trans_v[k // 8, bgl, k % 8, pl.ds(br, _L)] = vals


def _gather_kernel(table_hbm, idx_hbm, out_hbm,
                   idx_a, idx_b, rows_a, rows_b, trans_a, trans_b,
                   gsem_a, gsem_b, osem_a, osem_b):
    wid = lax.axis_index("s") * 2 + lax.axis_index("c")
    b0 = wid * _CB
    bg0 = wid * (_CB // 128)

    idx_bufs = (idx_a, idx_b)
    row_bufs = (rows_a, rows_b)
    trans_bufs = (trans_a, trans_b)
    gsems = (gsem_a, gsem_b)
    osems = (osem_a, osem_b)

    pltpu.sync_copy(idx_hbm.at[0, pl.ds(b0, _CB)], idx_a)
    pltpu.async_copy(table_hbm.at[idx_a], rows_a, gsem_a)

    @pl.loop(0, HIST_LEN, step=2)
    def _(l0):
        for p in range(2):
            l = l0 + p
            cur = p
            nxt = 1 - p

            @pl.when(l + 1 < HIST_LEN)
            def _():
                pltpu.sync_copy(idx_hbm.at[l + 1, pl.ds(b0, _CB)],
                                idx_bufs[nxt])
                pltpu.async_copy(table_hbm.at[idx_bufs[nxt]],
                                 row_bufs[nxt], gsems[nxt])

            pltpu.make_async_copy(table_hbm.at[idx_bufs[cur]],
                                  row_bufs[cur], gsems[cur]).wait()

            @pl.when(l >= 2)
            def _():
                pltpu.make_async_copy(
                    trans_bufs[cur],
                    out_hbm.at[l - 2, :, pl.ds(bg0, 4), :, :],
                    osems[cur],
                ).wait()

            _transpose_block(row_bufs[cur], trans_bufs[cur])
            pltpu.async_copy(trans_bufs[cur],
                             out_hbm.at[l, :, pl.ds(bg0, 4), :, :],
                             osems[cur])

    for l in (HIST_LEN - 2, HIST_LEN - 1):
        pltpu.make_async_copy(
            trans_bufs[l % 2],
            out_hbm.at[l, :, pl.ds(bg0, 4), :, :],
            osems[l % 2],
        ).wait()


@jax.jit
def _sc_gather(item_table, idx_t):
    mesh = plsc.VectorSubcoreMesh(core_axis_name="c", subcore_axis_name="s")
    return pl.kernel(
        _gather_kernel,
        out_type=jax.ShapeDtypeStruct((HIST_LEN, 4, BATCH // 128, 8, 128),
                                      jnp.float32),
        mesh=mesh,
        compiler_params=pltpu.CompilerParams(use_tc_tiling_on_sc=False,
                                             needs_layout_passes=False),
        scratch_types=[
            pltpu.VMEM((_CB,), jnp.int32),
            pltpu.VMEM((_CB,), jnp.int32),
            pltpu.VMEM((_CB, EMBEDDING_K), jnp.float32),
            pltpu.VMEM((_CB, EMBEDDING_K), jnp.float32),
            pltpu.VMEM((4, 4, 8, 128), jnp.float32),
            pltpu.VMEM((4, 4, 8, 128), jnp.float32),
            pltpu.SemaphoreType.DMA,
            pltpu.SemaphoreType.DMA,
            pltpu.SemaphoreType.DMA,
            pltpu.SemaphoreType.DMA,
        ],
    )(item_table, idx_t)


def kernel(item_table, item_idx):
    idx_t = item_idx.T.astype(jnp.int32)
    p = _sc_gather(item_table, idx_t)
    return p.transpose(2, 4, 0, 1, 3).reshape(BATCH, HIST_LEN, EMBEDDING_K)
